# baseline (speedup 1.0000x reference)
import functools

import numpy as np

import jax
import jax.numpy as jnp
from jax import lax
from jax.experimental import pallas as pl
from jax.experimental.pallas import tpu as pltpu
import jax.experimental.pallas.tpu_sc as plsc

NC = 2
NS = 16
NW = NC * NS
L = 16


def _worker_id():
    return lax.axis_index("c") * NS + lax.axis_index("s")


def kernel(h, action, policy, edge_index, W_fc, W_attn):
    N, IN_DIM = h.shape
    D = action.shape[1]
    E = edge_index.shape[1]
    f32 = jnp.float32

    src = edge_index[0].astype(jnp.int32)
    dst = edge_index[1].astype(jnp.int32)

    EPW = E // NW
    CHA = 2000
    DSPAN = 1000
    ZCH = 40
    assert E % NW == 0 and EPW % CHA == 0
    assert N % DSPAN == 0 and DSPAN % ZCH == 0 and N // DSPAN <= NS and N % L == 0

    a1 = W_attn[0, :D]
    a2 = W_attn[0, D:]
    Ap = jnp.zeros((D, 128), f32).at[:, 0].set(a1).at[:, 1].set(a2)
    Wt = W_fc.T

    BLK = 1000

    def _pre(h_ref, act_ref, pol_ref, Wt_ref, Ap_ref, S_ref, T_ref):
        U = jnp.dot(Wt_ref[...], Ap_ref[...], preferred_element_type=f32)
        S_ref[...] = jnp.dot(h_ref[...], U, preferred_element_type=f32)
        p = pol_ref[...]
        T_ref[:, 0, :] = p.astype(jnp.bfloat16)
        T_ref[:, 1, :] = (act_ref[...] - p).astype(jnp.bfloat16)

    S_pad, T = pl.pallas_call(
        _pre,
        grid=(N // BLK,),
        in_specs=[
            pl.BlockSpec((BLK, IN_DIM), lambda i: (i, 0)),
            pl.BlockSpec((BLK, D), lambda i: (i, 0)),
            pl.BlockSpec((BLK, D), lambda i: (i, 0)),
            pl.BlockSpec((IN_DIM, 128), lambda i: (0, 0)),
            pl.BlockSpec((128, 128), lambda i: (0, 0)),
        ],
        out_specs=[
            pl.BlockSpec((BLK, 128), lambda i: (i, 0)),
            pl.BlockSpec((BLK, 2, D), lambda i: (i, 0, 0)),
        ],
        out_shape=[
            jax.ShapeDtypeStruct((N, 128), f32),
            jax.ShapeDtypeStruct((N, 2, D), jnp.bfloat16),
        ],
    )(h, action, policy, Wt, Ap)
    s1 = S_pad[:, 0]
    s2 = S_pad[:, 1]

    mesh = plsc.VectorSubcoreMesh(core_axis_name="c", subcore_axis_name="s")

    WMAX = 5
    assert (CHA // L) % WMAX == 0

    def _edge_stats(src_hbm, dst_hbm, s1_hbm, s2_hbm, e_hbm, parts_hbm,
                    s1_v, s2_v, t0, t1, t2, t3, t4, src_c, dst_c, e_c):
        tabs = [t0, t1, t2, t3, t4]
        wid = _worker_id()
        base = wid * EPW
        pltpu.sync_copy(s1_hbm, s1_v)
        pltpu.sync_copy(s2_hbm, s2_v)

        @plsc.parallel_loop(0, N // L, unroll=4)
        def init(i):
            for t in range(WMAX):
                tabs[t][pl.ds(i * L, L)] = jnp.full((L,), -1e30, f32)

        def chunk(k, _):
            off = base + k * CHA
            pltpu.sync_copy(src_hbm.at[pl.ds(off, CHA)], src_c)
            pltpu.sync_copy(dst_hbm.at[pl.ds(off, CHA)], dst_c)

            @plsc.parallel_loop(0, CHA // L, unroll=4)
            def egrp(j):
                s16 = src_c[pl.ds(j * L, L)]
                d16 = dst_c[pl.ds(j * L, L)]
                e16 = plsc.load_gather(s1_v, [s16]) + plsc.load_gather(s2_v, [d16])
                e16 = jnp.where(e16 >= 0, e16, e16 * f32(0.01))
                e_c[pl.ds(j * L, L)] = e16

            def sgrp(g, _):
                ds_ = [dst_c[pl.ds((g * WMAX + t) * L, L)] for t in range(WMAX)]
                es = [e_c[pl.ds((g * WMAX + t) * L, L)] for t in range(WMAX)]

                def cond(pend):
                    m = pend[0]
                    for t in range(1, WMAX):
                        m = m | pend[t]
                    return jnp.any(m)

                def body(pend):
                    out = []
                    for t in range(WMAX):
                        cur = plsc.load_gather(tabs[t], [ds_[t]])
                        p = pend[t] & (es[t] > cur)
                        plsc.store_scatter(tabs[t], [ds_[t]], es[t], mask=p)
                        out.append(p)
                    return tuple(out)
                lax.while_loop(cond, body, (jnp.ones((L,), jnp.bool_),) * WMAX)
                return 0
            lax.fori_loop(0, CHA // (L * WMAX), sgrp, 0)
            pltpu.sync_copy(e_c, e_hbm.at[pl.ds(off, CHA)])
            return 0
        lax.fori_loop(0, EPW // CHA, chunk, 0)

        @plsc.parallel_loop(0, N // L, unroll=4)
        def merge(i):
            m = tabs[0][pl.ds(i * L, L)]
            for t in range(1, WMAX):
                m = jnp.maximum(m, tabs[t][pl.ds(i * L, L)])
            tabs[0][pl.ds(i * L, L)] = m
        pltpu.sync_copy(t0, parts_hbm.at[wid])

    e_arr, emax_parts = pl.kernel(
        _edge_stats,
        out_type=[
            jax.ShapeDtypeStruct((E,), f32),
            jax.ShapeDtypeStruct((NW, N), f32),
        ],
        mesh=mesh,
        compiler_params=pltpu.CompilerParams(needs_layout_passes=False),
        scratch_types=[
            pltpu.VMEM((N,), f32),
            pltpu.VMEM((N,), f32),
            pltpu.VMEM((N,), f32),
            pltpu.VMEM((N,), f32),
            pltpu.VMEM((N,), f32),
            pltpu.VMEM((N,), f32),
            pltpu.VMEM((N,), f32),
            pltpu.VMEM((CHA,), jnp.int32),
            pltpu.VMEM((CHA,), jnp.int32),
            pltpu.VMEM((CHA,), f32),
        ],
    )(src, dst, s1, s2)

    emax = pl.pallas_call(
        lambda p_ref, o_ref: o_ref.__setitem__(..., jnp.max(p_ref[...], axis=0)),
        out_shape=jax.ShapeDtypeStruct((N,), f32),
    )(emax_parts)

    def _edge_exp(dst_hbm, e_hbm, emax_hbm, ex_hbm, parts_hbm,
                  emax_v, den_v, dst_c, e_c, ex_c):
        wid = _worker_id()
        base = wid * EPW
        pltpu.sync_copy(emax_hbm, emax_v)

        def init(i, _):
            den_v[pl.ds(i * L, L)] = jnp.zeros((L,), f32)
            return 0
        lax.fori_loop(0, N // L, init, 0)

        def chunk(k, _):
            off = base + k * CHA
            pltpu.sync_copy(dst_hbm.at[pl.ds(off, CHA)], dst_c)
            pltpu.sync_copy(e_hbm.at[pl.ds(off, CHA)], e_c)

            @plsc.parallel_loop(0, CHA // L, unroll=4)
            def grp(j):
                d16 = dst_c[pl.ds(j * L, L)]
                e16 = e_c[pl.ds(j * L, L)]
                m16 = plsc.load_gather(emax_v, [d16])
                ex16 = jnp.exp(e16 - m16)
                ex_c[pl.ds(j * L, L)] = ex16
                plsc.addupdate_scatter(den_v, [d16], ex16)
            pltpu.sync_copy(ex_c, ex_hbm.at[pl.ds(off, CHA)])
            return 0
        lax.fori_loop(0, EPW // CHA, chunk, 0)
        pltpu.sync_copy(den_v, parts_hbm.at[wid])

    ex_arr, den_parts = pl.kernel(
        _edge_exp,
        out_type=[
            jax.ShapeDtypeStruct((E,), f32),
            jax.ShapeDtypeStruct((NW, N), f32),
        ],
        mesh=mesh,
        compiler_params=pltpu.CompilerParams(needs_layout_passes=False),
        scratch_types=[
            pltpu.VMEM((N,), f32),
            pltpu.VMEM((N,), f32),
            pltpu.VMEM((CHA,), jnp.int32),
            pltpu.VMEM((CHA,), f32),
            pltpu.VMEM((CHA,), f32),
        ],
    )(dst, e_arr, emax)

    denom = pl.pallas_call(
        lambda p_ref, o_ref: o_ref.__setitem__(..., jnp.sum(p_ref[...], axis=0)),
        out_shape=jax.ShapeDtypeStruct((N,), f32),
    )(den_parts)

    def _alpha(dst_hbm, ex_hbm, den_hbm, al_hbm, den_v, dst_c, ex_c, al_c):
        wid = _worker_id()
        base = wid * EPW
        pltpu.sync_copy(den_hbm, den_v)

        def chunk(k, _):
            off = base + k * CHA
            pltpu.sync_copy(dst_hbm.at[pl.ds(off, CHA)], dst_c)
            pltpu.sync_copy(ex_hbm.at[pl.ds(off, CHA)], ex_c)

            @plsc.parallel_loop(0, CHA // L, unroll=4)
            def grp(j):
                d16 = dst_c[pl.ds(j * L, L)]
                ex16 = ex_c[pl.ds(j * L, L)]
                dn16 = plsc.load_gather(den_v, [d16])
                al_c[pl.ds(j * L, L)] = ex16 / (dn16 + f32(1e-9))
            pltpu.sync_copy(al_c, al_hbm.at[pl.ds(off, CHA)])
            return 0
        lax.fori_loop(0, EPW // CHA, chunk, 0)

    alpha = pl.kernel(
        _alpha,
        out_type=jax.ShapeDtypeStruct((E,), f32),
        mesh=mesh,
        compiler_params=pltpu.CompilerParams(needs_layout_passes=False),
        scratch_types=[
            pltpu.VMEM((N,), f32),
            pltpu.VMEM((CHA,), jnp.int32),
            pltpu.VMEM((CHA,), f32),
            pltpu.VMEM((CHA,), f32),
        ],
    )(dst, ex_arr, denom)

    HC = 80
    BLKE = 2000
    UPB = BLKE // HC
    assert BLKE % HC == 0 and UPB % 2 == 1 and EPW % BLKE == 0
    dst3 = dst.reshape(E // BLKE, UPB, HC)
    T32 = lax.bitcast_convert_type(T.reshape(N, 2 * D // 2, 2), jnp.int32)

    def _aggregate(src_hbm, dst3_hbm, al_hbm, T_hbm, out_hbm,
                   src_b, dst_b, al_b, rows0, rows1, m0, m1, acc,
                   sem0, sem1, sems0, sems1):
        cid = lax.axis_index("c")
        sid = lax.axis_index("s")
        wid = cid * NS + sid
        base = wid * EPW

        def zinit(i, _):
            for b in range(D // L):
                m0[i, pl.ds(b * L, L)] = jnp.zeros((L,), f32)
            return 0
        lax.fori_loop(0, HC, zinit, 0)

        @pl.when(sid < N // DSPAN)
        def _zero():
            for k in range(DSPAN // ZCH):
                r0 = pl.multiple_of(sid * DSPAN + k * ZCH, 8)
                pltpu.sync_copy(m0.at[pl.ds(0, ZCH)], acc.at[pl.ds(r0, ZCH)])
        plsc.subcore_barrier()

        def _gather(u, rows, sem):
            off = pl.multiple_of(u * HC, 8)
            return pltpu.async_copy(T_hbm.at[src_b.at[pl.ds(off, HC)]], rows, sem)

        def _drain(u, rows, sem):
            off = pl.multiple_of(u * HC, 8)
            pltpu.make_async_copy(T_hbm.at[src_b.at[pl.ds(off, HC)]], rows, sem).wait()

        def _consume(u, rows, m_v, sems, first):
            jbase = u * HC

            @pl.when(jnp.logical_not(first))
            def _wait_prev():
                pltpu.make_async_copy(m_v, acc.at[dst_b.at[u]], sems).wait()

            @plsc.parallel_loop(0, HC, unroll=4)
            def _edges(i):
                al = plsc.load_gather(al_b, [jnp.broadcast_to(jbase + i, (L,))])
                for b in range(D // (2 * L)):
                    pv = plsc.bitcast(rows[i, pl.ds(b * L, L)], jnp.bfloat16)
                    dv = plsc.bitcast(rows[i, pl.ds(D // 2 + b * L, L)], jnp.bfloat16)
                    p_e, p_o = plsc.unpack(pv, format=plsc.PackFormat.INTERLEAVED,
                                           preferred_element_type=f32)
                    d_e, d_o = plsc.unpack(dv, format=plsc.PackFormat.INTERLEAVED,
                                           preferred_element_type=f32)
                    m_v[i, pl.ds(b * 2 * L, L)] = p_e + al * d_e
                    m_v[i, pl.ds(b * 2 * L + L, L)] = p_o + al * d_o
            pltpu.async_copy(m_v, acc.at[dst_b.at[u]], sems, add=True)

        def block(bk, _):
            eoff = base + bk * BLKE
            gblk = wid * (EPW // BLKE) + bk
            pltpu.sync_copy(src_hbm.at[pl.ds(eoff, BLKE)], src_b)
            pltpu.sync_copy(dst3_hbm.at[gblk], dst_b)
            pltpu.sync_copy(al_hbm.at[pl.ds(eoff, BLKE)], al_b)
            _gather(0, rows0, sem0)

            def pair(k, _):
                first = jnp.logical_and(bk == 0, k == 0)
                _gather(2 * k + 1, rows1, sem1)
                _drain(2 * k, rows0, sem0)
                _consume(2 * k, rows0, m0, sems0, first)
                _gather(2 * k + 2, rows0, sem0)
                _drain(2 * k + 1, rows1, sem1)
                _consume(2 * k + 1, rows1, m1, sems1, first)
                return 0
            lax.fori_loop(0, (UPB - 1) // 2, pair, 0)
            _drain(UPB - 1, rows0, sem0)
            _consume(UPB - 1, rows0, m0, sems0, jnp.bool_(False))
            return 0
        lax.fori_loop(0, EPW // BLKE, block, 0)
        pltpu.make_async_copy(m0, acc.at[dst_b.at[0]], sems0).wait()
        pltpu.make_async_copy(m1, acc.at[dst_b.at[0]], sems1).wait()
        plsc.subcore_barrier()

        @pl.when(sid < N // DSPAN)
        def _dump():
            r0 = pl.multiple_of(sid * DSPAN, 8)
            pltpu.sync_copy(acc.at[pl.ds(r0, DSPAN)], out_hbm.at[cid, pl.ds(r0, DSPAN)])

    out_parts = pl.kernel(
        _aggregate,
        out_type=jax.ShapeDtypeStruct((NC, N, D), f32),
        mesh=mesh,
        compiler_params=pltpu.CompilerParams(needs_layout_passes=False),
        scratch_types=[
            pltpu.VMEM((BLKE,), jnp.int32),
            pltpu.VMEM((UPB, HC), jnp.int32),
            pltpu.VMEM((BLKE,), f32),
            pltpu.VMEM((HC, D), jnp.int32),
            pltpu.VMEM((HC, D), jnp.int32),
            pltpu.VMEM((HC, D), f32),
            pltpu.VMEM((HC, D), f32),
            pltpu.VMEM_SHARED((N, D), f32),
            pltpu.SemaphoreType.DMA,
            pltpu.SemaphoreType.DMA,
            pltpu.SemaphoreType.DMA,
            pltpu.SemaphoreType.DMA,
        ],
    )(src, dst3, alpha, T32)

    out_m = pl.pallas_call(
        lambda p_ref, o_ref: o_ref.__setitem__(..., p_ref[0] + p_ref[1]),
        out_shape=jax.ShapeDtypeStruct((N, D), f32),
    )(out_parts)
    perm = np.empty((D,), np.int32)
    for b in range(D // 32):
        for j in range(16):
            perm[32 * b + 2 * j] = 32 * b + j
            perm[32 * b + 2 * j + 1] = 32 * b + 16 + j
    return jnp.take(out_m, jnp.asarray(perm), axis=1)


# speedup vs baseline: 1.3816x; 1.3816x over previous
import functools

import numpy as np

import jax
import jax.numpy as jnp
from jax import lax
from jax.experimental import pallas as pl
from jax.experimental.pallas import tpu as pltpu
import jax.experimental.pallas.tpu_sc as plsc

NC = 2
NS = 16
NW = NC * NS
L = 16


def _worker_id():
    return lax.axis_index("c") * NS + lax.axis_index("s")


def kernel(h, action, policy, edge_index, W_fc, W_attn):
    N, IN_DIM = h.shape
    D = action.shape[1]
    E = edge_index.shape[1]
    f32 = jnp.float32

    src = edge_index[0].astype(jnp.int32)
    dst = edge_index[1].astype(jnp.int32)

    EPW = E // NW
    CHA = 2000
    DSPAN = 1000
    ZCH = 40
    assert E % NW == 0 and EPW % CHA == 0
    assert N % DSPAN == 0 and DSPAN % ZCH == 0 and N // DSPAN <= NS and N % L == 0

    a1 = W_attn[0, :D]
    a2 = W_attn[0, D:]
    Ap = jnp.zeros((D, 128), f32).at[:, 0].set(a1).at[:, 1].set(a2)
    Wt = W_fc.T

    BLK = 1000

    def _pre(h_ref, act_ref, pol_ref, Wt_ref, Ap_ref, S_ref, T_ref):
        U = jnp.dot(Wt_ref[...], Ap_ref[...], preferred_element_type=f32)
        S_ref[...] = jnp.dot(h_ref[...], U, preferred_element_type=f32)
        p = pol_ref[...]
        d = act_ref[...] - p
        pu = lax.convert_element_type(
            lax.bitcast_convert_type(p.astype(jnp.bfloat16), jnp.uint16), jnp.uint32)
        du = lax.convert_element_type(
            lax.bitcast_convert_type(d.astype(jnp.bfloat16), jnp.uint16), jnp.uint32)
        T_ref[...] = (du << 16) | pu

    S_pad, T = pl.pallas_call(
        _pre,
        grid=(N // BLK,),
        in_specs=[
            pl.BlockSpec((BLK, IN_DIM), lambda i: (i, 0)),
            pl.BlockSpec((BLK, D), lambda i: (i, 0)),
            pl.BlockSpec((BLK, D), lambda i: (i, 0)),
            pl.BlockSpec((IN_DIM, 128), lambda i: (0, 0)),
            pl.BlockSpec((128, 128), lambda i: (0, 0)),
        ],
        out_specs=[
            pl.BlockSpec((BLK, 128), lambda i: (i, 0)),
            pl.BlockSpec((BLK, D), lambda i: (i, 0)),
        ],
        out_shape=[
            jax.ShapeDtypeStruct((N, 128), f32),
            jax.ShapeDtypeStruct((N, D), jnp.uint32),
        ],
    )(h, action, policy, Wt, Ap)
    s1 = S_pad[:, 0]
    s2 = S_pad[:, 1]

    mesh = plsc.VectorSubcoreMesh(core_axis_name="c", subcore_axis_name="s")

    WMAX = 5
    assert (CHA // L) % WMAX == 0

    def _edge_stats(src_hbm, dst_hbm, s1_hbm, s2_hbm, e_hbm, parts_hbm,
                    s1_v, s2_v, t0, t1, t2, t3, t4, src_c, dst_c, e_c):
        tabs = [t0, t1, t2, t3, t4]
        wid = _worker_id()
        base = wid * EPW
        pltpu.sync_copy(s1_hbm, s1_v)
        pltpu.sync_copy(s2_hbm, s2_v)

        @plsc.parallel_loop(0, N // L, unroll=4)
        def init(i):
            for t in range(WMAX):
                tabs[t][pl.ds(i * L, L)] = jnp.full((L,), -1e30, f32)

        def chunk(k, _):
            off = base + k * CHA
            pltpu.sync_copy(src_hbm.at[pl.ds(off, CHA)], src_c)
            pltpu.sync_copy(dst_hbm.at[pl.ds(off, CHA)], dst_c)

            @plsc.parallel_loop(0, CHA // L, unroll=4)
            def egrp(j):
                s16 = src_c[pl.ds(j * L, L)]
                d16 = dst_c[pl.ds(j * L, L)]
                e16 = plsc.load_gather(s1_v, [s16]) + plsc.load_gather(s2_v, [d16])
                e16 = jnp.where(e16 >= 0, e16, e16 * f32(0.01))
                e_c[pl.ds(j * L, L)] = e16

            def sgrp(g, _):
                ds_ = [dst_c[pl.ds((g * WMAX + t) * L, L)] for t in range(WMAX)]
                es = [e_c[pl.ds((g * WMAX + t) * L, L)] for t in range(WMAX)]

                def cond(pend):
                    m = pend[0]
                    for t in range(1, WMAX):
                        m = m | pend[t]
                    return jnp.any(m)

                def body(pend):
                    out = []
                    for t in range(WMAX):
                        cur = plsc.load_gather(tabs[t], [ds_[t]])
                        p = pend[t] & (es[t] > cur)
                        plsc.store_scatter(tabs[t], [ds_[t]], es[t], mask=p)
                        out.append(p)
                    return tuple(out)
                lax.while_loop(cond, body, (jnp.ones((L,), jnp.bool_),) * WMAX)
                return 0
            lax.fori_loop(0, CHA // (L * WMAX), sgrp, 0)
            pltpu.sync_copy(e_c, e_hbm.at[pl.ds(off, CHA)])
            return 0
        lax.fori_loop(0, EPW // CHA, chunk, 0)

        @plsc.parallel_loop(0, N // L, unroll=4)
        def merge(i):
            m = tabs[0][pl.ds(i * L, L)]
            for t in range(1, WMAX):
                m = jnp.maximum(m, tabs[t][pl.ds(i * L, L)])
            tabs[0][pl.ds(i * L, L)] = m
        pltpu.sync_copy(t0, parts_hbm.at[wid])

    e_arr, emax_parts = pl.kernel(
        _edge_stats,
        out_type=[
            jax.ShapeDtypeStruct((E,), f32),
            jax.ShapeDtypeStruct((NW, N), f32),
        ],
        mesh=mesh,
        compiler_params=pltpu.CompilerParams(needs_layout_passes=False),
        scratch_types=[
            pltpu.VMEM((N,), f32),
            pltpu.VMEM((N,), f32),
            pltpu.VMEM((N,), f32),
            pltpu.VMEM((N,), f32),
            pltpu.VMEM((N,), f32),
            pltpu.VMEM((N,), f32),
            pltpu.VMEM((N,), f32),
            pltpu.VMEM((CHA,), jnp.int32),
            pltpu.VMEM((CHA,), jnp.int32),
            pltpu.VMEM((CHA,), f32),
        ],
    )(src, dst, s1, s2)

    emax = pl.pallas_call(
        lambda p_ref, o_ref: o_ref.__setitem__(..., jnp.max(p_ref[...], axis=0)),
        out_shape=jax.ShapeDtypeStruct((N,), f32),
    )(emax_parts)

    def _edge_exp(dst_hbm, e_hbm, emax_hbm, ex_hbm, parts_hbm,
                  emax_v, den_v, dst_c, e_c, ex_c):
        wid = _worker_id()
        base = wid * EPW
        pltpu.sync_copy(emax_hbm, emax_v)

        def init(i, _):
            den_v[pl.ds(i * L, L)] = jnp.zeros((L,), f32)
            return 0
        lax.fori_loop(0, N // L, init, 0)

        def chunk(k, _):
            off = base + k * CHA
            pltpu.sync_copy(dst_hbm.at[pl.ds(off, CHA)], dst_c)
            pltpu.sync_copy(e_hbm.at[pl.ds(off, CHA)], e_c)

            @plsc.parallel_loop(0, CHA // L, unroll=4)
            def grp(j):
                d16 = dst_c[pl.ds(j * L, L)]
                e16 = e_c[pl.ds(j * L, L)]
                m16 = plsc.load_gather(emax_v, [d16])
                ex16 = jnp.exp(e16 - m16)
                ex_c[pl.ds(j * L, L)] = ex16
                plsc.addupdate_scatter(den_v, [d16], ex16)
            pltpu.sync_copy(ex_c, ex_hbm.at[pl.ds(off, CHA)])
            return 0
        lax.fori_loop(0, EPW // CHA, chunk, 0)
        pltpu.sync_copy(den_v, parts_hbm.at[wid])

    ex_arr, den_parts = pl.kernel(
        _edge_exp,
        out_type=[
            jax.ShapeDtypeStruct((E,), f32),
            jax.ShapeDtypeStruct((NW, N), f32),
        ],
        mesh=mesh,
        compiler_params=pltpu.CompilerParams(needs_layout_passes=False),
        scratch_types=[
            pltpu.VMEM((N,), f32),
            pltpu.VMEM((N,), f32),
            pltpu.VMEM((CHA,), jnp.int32),
            pltpu.VMEM((CHA,), f32),
            pltpu.VMEM((CHA,), f32),
        ],
    )(dst, e_arr, emax)

    denom = pl.pallas_call(
        lambda p_ref, o_ref: o_ref.__setitem__(..., jnp.sum(p_ref[...], axis=0)),
        out_shape=jax.ShapeDtypeStruct((N,), f32),
    )(den_parts)

    def _alpha(dst_hbm, ex_hbm, den_hbm, al_hbm, den_v, dst_c, ex_c, al_c):
        wid = _worker_id()
        base = wid * EPW
        pltpu.sync_copy(den_hbm, den_v)

        def chunk(k, _):
            off = base + k * CHA
            pltpu.sync_copy(dst_hbm.at[pl.ds(off, CHA)], dst_c)
            pltpu.sync_copy(ex_hbm.at[pl.ds(off, CHA)], ex_c)

            @plsc.parallel_loop(0, CHA // L, unroll=4)
            def grp(j):
                d16 = dst_c[pl.ds(j * L, L)]
                ex16 = ex_c[pl.ds(j * L, L)]
                dn16 = plsc.load_gather(den_v, [d16])
                al_c[pl.ds(j * L, L)] = ex16 / (dn16 + f32(1e-9))
            pltpu.sync_copy(al_c, al_hbm.at[pl.ds(off, CHA)])
            return 0
        lax.fori_loop(0, EPW // CHA, chunk, 0)

    alpha = pl.kernel(
        _alpha,
        out_type=jax.ShapeDtypeStruct((E,), f32),
        mesh=mesh,
        compiler_params=pltpu.CompilerParams(needs_layout_passes=False),
        scratch_types=[
            pltpu.VMEM((N,), f32),
            pltpu.VMEM((CHA,), jnp.int32),
            pltpu.VMEM((CHA,), f32),
            pltpu.VMEM((CHA,), f32),
        ],
    )(dst, ex_arr, denom)

    HC = 80
    BLKE = 2000
    UPB = BLKE // HC
    assert BLKE % HC == 0 and UPB % 2 == 1 and EPW % BLKE == 0
    dst3 = dst.reshape(E // BLKE, UPB, HC)

    def _aggregate(src_hbm, dst3_hbm, al_hbm, T_hbm, out_hbm,
                   src_b, dst_b, al_b, rows0, rows1, m0, m1, acc,
                   sem0, sem1, sems0, sems1):
        cid = lax.axis_index("c")
        sid = lax.axis_index("s")
        wid = cid * NS + sid
        base = wid * EPW

        def zinit(i, _):
            for b in range(D // L):
                m0[i, pl.ds(b * L, L)] = jnp.zeros((L,), f32)
            return 0
        lax.fori_loop(0, HC, zinit, 0)

        @pl.when(sid < N // DSPAN)
        def _zero():
            for k in range(DSPAN // ZCH):
                r0 = pl.multiple_of(sid * DSPAN + k * ZCH, 8)
                pltpu.sync_copy(m0.at[pl.ds(0, ZCH)], acc.at[pl.ds(r0, ZCH)])
        plsc.subcore_barrier()

        def _gather(u, rows, sem):
            off = pl.multiple_of(u * HC, 8)
            return pltpu.async_copy(T_hbm.at[src_b.at[pl.ds(off, HC)]], rows, sem)

        def _drain(u, rows, sem):
            off = pl.multiple_of(u * HC, 8)
            pltpu.make_async_copy(T_hbm.at[src_b.at[pl.ds(off, HC)]], rows, sem).wait()

        def _consume(u, rows, m_v, sems, first):
            jbase = u * HC

            @pl.when(jnp.logical_not(first))
            def _wait_prev():
                pltpu.make_async_copy(m_v, acc.at[dst_b.at[u]], sems).wait()

            @plsc.parallel_loop(0, HC, unroll=4)
            def _edges(i):
                al = plsc.load_gather(al_b, [jnp.broadcast_to(jbase + i, (L,))])
                for b in range(D // L):
                    pd = plsc.bitcast(rows[i, pl.ds(b * L, L)], jnp.bfloat16)
                    p16, d16 = plsc.unpack(pd, format=plsc.PackFormat.INTERLEAVED,
                                           preferred_element_type=f32)
                    m_v[i, pl.ds(b * L, L)] = p16 + al * d16
            pltpu.async_copy(m_v, acc.at[dst_b.at[u]], sems, add=True)

        def block(bk, _):
            eoff = base + bk * BLKE
            gblk = wid * (EPW // BLKE) + bk
            pltpu.sync_copy(src_hbm.at[pl.ds(eoff, BLKE)], src_b)
            pltpu.sync_copy(dst3_hbm.at[gblk], dst_b)
            pltpu.sync_copy(al_hbm.at[pl.ds(eoff, BLKE)], al_b)
            _gather(0, rows0, sem0)

            def pair(k, _):
                first = jnp.logical_and(bk == 0, k == 0)
                _gather(2 * k + 1, rows1, sem1)
                _drain(2 * k, rows0, sem0)
                _consume(2 * k, rows0, m0, sems0, first)
                _gather(2 * k + 2, rows0, sem0)
                _drain(2 * k + 1, rows1, sem1)
                _consume(2 * k + 1, rows1, m1, sems1, first)
                return 0
            lax.fori_loop(0, (UPB - 1) // 2, pair, 0)
            _drain(UPB - 1, rows0, sem0)
            _consume(UPB - 1, rows0, m0, sems0, jnp.bool_(False))
            return 0
        lax.fori_loop(0, EPW // BLKE, block, 0)
        pltpu.make_async_copy(m0, acc.at[dst_b.at[0]], sems0).wait()
        pltpu.make_async_copy(m1, acc.at[dst_b.at[0]], sems1).wait()
        plsc.subcore_barrier()

        @pl.when(sid < N // DSPAN)
        def _dump():
            r0 = pl.multiple_of(sid * DSPAN, 8)
            pltpu.sync_copy(acc.at[pl.ds(r0, DSPAN)], out_hbm.at[cid, pl.ds(r0, DSPAN)])

    out_parts = pl.kernel(
        _aggregate,
        out_type=jax.ShapeDtypeStruct((NC, N, D), f32),
        mesh=mesh,
        compiler_params=pltpu.CompilerParams(needs_layout_passes=False),
        scratch_types=[
            pltpu.VMEM((BLKE,), jnp.int32),
            pltpu.VMEM((UPB, HC), jnp.int32),
            pltpu.VMEM((BLKE,), f32),
            pltpu.VMEM((HC, D), jnp.uint32),
            pltpu.VMEM((HC, D), jnp.uint32),
            pltpu.VMEM((HC, D), f32),
            pltpu.VMEM((HC, D), f32),
            pltpu.VMEM_SHARED((N, D), f32),
            pltpu.SemaphoreType.DMA,
            pltpu.SemaphoreType.DMA,
            pltpu.SemaphoreType.DMA,
            pltpu.SemaphoreType.DMA,
        ],
    )(src, dst3, alpha, T)

    out = pl.pallas_call(
        lambda p_ref, o_ref: o_ref.__setitem__(..., p_ref[0] + p_ref[1]),
        out_shape=jax.ShapeDtypeStruct((N, D), f32),
    )(out_parts)
    return out
